# baseline (speedup 1.0000x reference)
import functools

import jax
import jax.numpy as jnp
from jax import lax
from jax.experimental import pallas as pl
from jax.experimental.pallas import tpu as pltpu
from jax.experimental.pallas import tpu_sc as plsc

NUM_EXPERTS = 64
TOP_K = 8
SCALING = 32 / 8

_TM = 2048
_LANES = 16
_NW = 32
_NCHUNK = 2


def _tc1_body(x_ref, aq_ref, k_ref, z_ref, s_ref):
    y = jnp.dot(x_ref[...], aq_ref[...], preferred_element_type=jnp.float32)
    z_ref[...] = y[:, :NUM_EXPERTS]
    q = y[:, NUM_EXPERTS:]
    st = jnp.dot(k_ref[...], q.T, preferred_element_type=jnp.float32)
    ntok = s_ref.shape[2]
    nsub = y.shape[0] // ntok
    s_ref[...] = jnp.stack(
        [st[:, k * ntok : (k + 1) * ntok] for k in range(nsub)], axis=0
    )


def _sc_route_body(s_hbm, w_hbm, i_hbm, svm, wvm, ivm):
    tok = svm.shape[0] // NUM_EXPERTS
    wid = lax.axis_index("s") * 2 + lax.axis_index("c")
    chunk = NUM_EXPERTS * tok
    pltpu.sync_copy(s_hbm.at[pl.ds(wid * chunk, chunk)], svm)

    neg = jnp.full((_LANES,), -jnp.inf, jnp.float32)
    zero_i = jnp.zeros((_LANES,), jnp.int32)

    def group_body(g, acc):
        off = g * _LANES

        def e_body(e, carry):
            ts = list(carry[:TOP_K])
            js = list(carry[TOP_K:])
            v = svm[pl.ds(e * tok + off, _LANES)]
            ve = jnp.full((_LANES,), e, jnp.int32)
            cs = [v > ts[j] for j in range(TOP_K)]
            nts = [jnp.where(cs[0], v, ts[0])]
            njs = [jnp.where(cs[0], ve, js[0])]
            for j in range(1, TOP_K):
                nts.append(jnp.where(cs[j], jnp.where(cs[j - 1], ts[j - 1], v), ts[j]))
                njs.append(jnp.where(cs[j], jnp.where(cs[j - 1], js[j - 1], ve), js[j]))
            return tuple(nts) + tuple(njs)

        init = (neg,) * TOP_K + (zero_i,) * TOP_K
        res = lax.fori_loop(0, NUM_EXPERTS, e_body, init)
        ts = res[:TOP_K]
        js = res[TOP_K:]
        m0 = ts[0]
        es = [jnp.exp(t - m0) for t in ts]
        ssum = es[0]
        for j in range(1, TOP_K):
            ssum = ssum + es[j]
        inv = 1.0 / ssum
        for j in range(TOP_K):
            wvm[pl.ds(j * tok + off, _LANES)] = es[j] * inv
            ivm[pl.ds(j * tok + off, _LANES)] = js[j]
        return acc

    lax.fori_loop(0, tok // _LANES, group_body, 0)
    ochunk = TOP_K * tok
    pltpu.sync_copy(wvm, w_hbm.at[pl.ds(wid * ochunk, ochunk)])
    pltpu.sync_copy(ivm, i_hbm.at[pl.ds(wid * ochunk, ochunk)])


def _tc2_body(z_ref, w_ref, i_ref, b_ref, o_ref):
    nsub, _, ntok = w_ref.shape
    iota = lax.broadcasted_iota(jnp.int32, (NUM_EXPERTS, ntok), 0)
    parts = []
    for k in range(nsub):
        gt = jnp.zeros((NUM_EXPERTS, ntok), jnp.float32)
        for j in range(TOP_K):
            ij = i_ref[k, j, :].reshape(1, ntok)
            wj = w_ref[k, j, :].reshape(1, ntok)
            gt = gt + jnp.where(iota == ij, wj, 0.0)
        parts.append(gt.T)
    g = jnp.concatenate(parts, axis=0)
    zg = z_ref[...] * g
    o_ref[...] = jnp.dot(zg, b_ref[...], preferred_element_type=jnp.float32) * SCALING


@jax.jit
def kernel(x, A_w, W_query_w, keys, B_w):
    bsz, ssz, in_f = x.shape
    out_f = B_w.shape[1]
    t = bsz * ssz
    xf = x.reshape(t, in_f)
    aq = jnp.concatenate([A_w, W_query_w], axis=1)

    tchunk = t // _NCHUNK
    tok = tchunk // _NW
    blocks_per_chunk = tchunk // _TM
    nsub_per_block = _TM // tok

    mesh = plsc.VectorSubcoreMesh(core_axis_name="c", subcore_axis_name="s")
    route = functools.partial(
        pl.kernel,
        out_type=[
            jax.ShapeDtypeStruct((_NW * TOP_K * tok,), jnp.float32),
            jax.ShapeDtypeStruct((_NW * TOP_K * tok,), jnp.int32),
        ],
        mesh=mesh,
        scratch_types=[
            pltpu.VMEM((NUM_EXPERTS * tok,), jnp.float32),
            pltpu.VMEM((TOP_K * tok,), jnp.float32),
            pltpu.VMEM((TOP_K * tok,), jnp.int32),
        ],
    )(_sc_route_body)

    z_parts, w_parts, i_parts = [], [], []
    for c in range(_NCHUNK):
        z_c, s_c = pl.pallas_call(
            _tc1_body,
            grid=(blocks_per_chunk,),
            in_specs=[
                pl.BlockSpec((_TM, in_f), lambda i, c=c: (c * blocks_per_chunk + i, 0)),
                pl.BlockSpec(aq.shape, lambda i: (0, 0)),
                pl.BlockSpec(keys.shape, lambda i: (0, 0)),
            ],
            out_specs=[
                pl.BlockSpec((_TM, NUM_EXPERTS), lambda i: (i, 0)),
                pl.BlockSpec((nsub_per_block, NUM_EXPERTS, tok), lambda i: (i, 0, 0)),
            ],
            out_shape=[
                jax.ShapeDtypeStruct((tchunk, NUM_EXPERTS), jnp.float32),
                jax.ShapeDtypeStruct((_NW, NUM_EXPERTS, tok), jnp.float32),
            ],
        )(xf, aq, keys)
        w_c, i_c = route(s_c.reshape(_NW * NUM_EXPERTS * tok))
        z_parts.append(z_c)
        w_parts.append(w_c)
        i_parts.append(i_c)

    z = jnp.concatenate(z_parts, axis=0)
    w = jnp.concatenate(w_parts, axis=0).reshape(_NCHUNK * _NW, TOP_K, tok)
    i = jnp.concatenate(i_parts, axis=0).reshape(_NCHUNK * _NW, TOP_K, tok)

    out = pl.pallas_call(
        _tc2_body,
        grid=(t // _TM,),
        in_specs=[
            pl.BlockSpec((_TM, NUM_EXPERTS), lambda i: (i, 0)),
            pl.BlockSpec((nsub_per_block, TOP_K, tok), lambda i: (i, 0, 0)),
            pl.BlockSpec((nsub_per_block, TOP_K, tok), lambda i: (i, 0, 0)),
            pl.BlockSpec(B_w.shape, lambda i: (0, 0)),
        ],
        out_specs=pl.BlockSpec((_TM, out_f), lambda i: (i, 0)),
        out_shape=jax.ShapeDtypeStruct((t, out_f), jnp.float32),
    )(z, w, i, B_w)
    return out.reshape(bsz, ssz, out_f)


# speedup vs baseline: 1.0358x; 1.0358x over previous
import functools

import jax
import jax.numpy as jnp
from jax import lax
from jax.experimental import pallas as pl
from jax.experimental.pallas import tpu as pltpu
from jax.experimental.pallas import tpu_sc as plsc

NUM_EXPERTS = 64
TOP_K = 8
SCALING = 32 / 8

_TM = 2048
_LANES = 16
_NW = 32
_NCHUNK = 4


def _tc1_body(x_ref, aq_ref, k_ref, z_ref, s_ref):
    y = jnp.dot(x_ref[...], aq_ref[...], preferred_element_type=jnp.float32)
    z_ref[...] = y[:, :NUM_EXPERTS]
    q = y[:, NUM_EXPERTS:]
    st = jnp.dot(k_ref[...], q.T, preferred_element_type=jnp.float32)
    ntok = s_ref.shape[2]
    nsub = y.shape[0] // ntok
    s_ref[...] = jnp.stack(
        [st[:, k * ntok : (k + 1) * ntok] for k in range(nsub)], axis=0
    )


def _sc_route_body(s_hbm, w_hbm, i_hbm, svm, wvm, ivm):
    tok = svm.shape[0] // NUM_EXPERTS
    wid = lax.axis_index("s") * 2 + lax.axis_index("c")
    chunk = NUM_EXPERTS * tok
    pltpu.sync_copy(s_hbm.at[pl.ds(wid * chunk, chunk)], svm)

    neg = jnp.full((_LANES,), -jnp.inf, jnp.float32)
    zero_i = jnp.zeros((_LANES,), jnp.int32)

    def group_body(g, acc):
        off = g * _LANES

        unroll = 4

        def e_body(eu, carry):
            ts = list(carry[:TOP_K])
            js = list(carry[TOP_K:])
            for u in range(unroll):
                e = eu * unroll + u
                v = svm[pl.ds(e * tok + off, _LANES)]
                ve = jnp.full((_LANES,), e, jnp.int32)
                cs = [v > ts[j] for j in range(TOP_K)]
                nts = [jnp.where(cs[0], v, ts[0])]
                njs = [jnp.where(cs[0], ve, js[0])]
                for j in range(1, TOP_K):
                    nts.append(
                        jnp.where(cs[j], jnp.where(cs[j - 1], ts[j - 1], v), ts[j])
                    )
                    njs.append(
                        jnp.where(cs[j], jnp.where(cs[j - 1], js[j - 1], ve), js[j])
                    )
                ts, js = nts, njs
            return tuple(ts) + tuple(js)

        init = (neg,) * TOP_K + (zero_i,) * TOP_K
        res = lax.fori_loop(0, NUM_EXPERTS // unroll, e_body, init)
        ts = res[:TOP_K]
        js = res[TOP_K:]
        m0 = ts[0]
        es = [jnp.exp(t - m0) for t in ts]
        ssum = es[0]
        for j in range(1, TOP_K):
            ssum = ssum + es[j]
        inv = 1.0 / ssum
        for j in range(TOP_K):
            wvm[pl.ds(j * tok + off, _LANES)] = es[j] * inv
            ivm[pl.ds(j * tok + off, _LANES)] = js[j]
        return acc

    lax.fori_loop(0, tok // _LANES, group_body, 0)
    ochunk = TOP_K * tok
    pltpu.sync_copy(wvm, w_hbm.at[pl.ds(wid * ochunk, ochunk)])
    pltpu.sync_copy(ivm, i_hbm.at[pl.ds(wid * ochunk, ochunk)])


def _tc2_body(z_ref, w_ref, i_ref, b_ref, o_ref):
    nsub, _, ntok = w_ref.shape
    iota = lax.broadcasted_iota(jnp.int32, (NUM_EXPERTS, ntok), 0)
    parts = []
    for k in range(nsub):
        gt = jnp.zeros((NUM_EXPERTS, ntok), jnp.float32)
        for j in range(TOP_K):
            ij = i_ref[k, j, :].reshape(1, ntok)
            wj = w_ref[k, j, :].reshape(1, ntok)
            gt = gt + jnp.where(iota == ij, wj, 0.0)
        parts.append(gt.T)
    g = jnp.concatenate(parts, axis=0)
    zg = z_ref[...] * g
    o_ref[...] = jnp.dot(zg, b_ref[...], preferred_element_type=jnp.float32) * SCALING


@jax.jit
def kernel(x, A_w, W_query_w, keys, B_w):
    bsz, ssz, in_f = x.shape
    out_f = B_w.shape[1]
    t = bsz * ssz
    xf = x.reshape(t, in_f)
    aq = jnp.concatenate([A_w, W_query_w], axis=1)

    tchunk = t // _NCHUNK
    tok = tchunk // _NW
    blocks_per_chunk = tchunk // _TM
    nsub_per_block = _TM // tok

    mesh = plsc.VectorSubcoreMesh(core_axis_name="c", subcore_axis_name="s")
    route = functools.partial(
        pl.kernel,
        out_type=[
            jax.ShapeDtypeStruct((_NW * TOP_K * tok,), jnp.float32),
            jax.ShapeDtypeStruct((_NW * TOP_K * tok,), jnp.int32),
        ],
        mesh=mesh,
        scratch_types=[
            pltpu.VMEM((NUM_EXPERTS * tok,), jnp.float32),
            pltpu.VMEM((TOP_K * tok,), jnp.float32),
            pltpu.VMEM((TOP_K * tok,), jnp.int32),
        ],
    )(_sc_route_body)

    z_parts, w_parts, i_parts = [], [], []
    for c in range(_NCHUNK):
        z_c, s_c = pl.pallas_call(
            _tc1_body,
            grid=(blocks_per_chunk,),
            in_specs=[
                pl.BlockSpec((_TM, in_f), lambda i, c=c: (c * blocks_per_chunk + i, 0)),
                pl.BlockSpec(aq.shape, lambda i: (0, 0)),
                pl.BlockSpec(keys.shape, lambda i: (0, 0)),
            ],
            out_specs=[
                pl.BlockSpec((_TM, NUM_EXPERTS), lambda i: (i, 0)),
                pl.BlockSpec((nsub_per_block, NUM_EXPERTS, tok), lambda i: (i, 0, 0)),
            ],
            out_shape=[
                jax.ShapeDtypeStruct((tchunk, NUM_EXPERTS), jnp.float32),
                jax.ShapeDtypeStruct((_NW, NUM_EXPERTS, tok), jnp.float32),
            ],
        )(xf, aq, keys)
        w_c, i_c = route(s_c.reshape(_NW * NUM_EXPERTS * tok))
        z_parts.append(z_c)
        w_parts.append(w_c)
        i_parts.append(i_c)

    z = jnp.concatenate(z_parts, axis=0)
    w = jnp.concatenate(w_parts, axis=0).reshape(_NCHUNK * _NW, TOP_K, tok)
    i = jnp.concatenate(i_parts, axis=0).reshape(_NCHUNK * _NW, TOP_K, tok)

    out = pl.pallas_call(
        _tc2_body,
        grid=(t // _TM,),
        in_specs=[
            pl.BlockSpec((_TM, NUM_EXPERTS), lambda i: (i, 0)),
            pl.BlockSpec((nsub_per_block, TOP_K, tok), lambda i: (i, 0, 0)),
            pl.BlockSpec((nsub_per_block, TOP_K, tok), lambda i: (i, 0, 0)),
            pl.BlockSpec(B_w.shape, lambda i: (0, 0)),
        ],
        out_specs=pl.BlockSpec((_TM, out_f), lambda i: (i, 0)),
        out_shape=jax.ShapeDtypeStruct((t, out_f), jnp.float32),
    )(z, w, i, B_w)
    return out.reshape(bsz, ssz, out_f)
